# Initial kernel scaffold; baseline (speedup 1.0000x reference)
import jax
import jax.numpy as jnp
from jax.experimental import pallas as pl


def kernel(x, edge_index, W1, b1, W2, b2, Wc1, bc1, Wc2, bc2):
    raise NotImplementedError("write your pallas kernel here")


# speedup vs baseline: 8.0648x; 8.0648x over previous
import functools

import jax
import jax.numpy as jnp
from jax import lax
from jax.experimental import pallas as pl
from jax.experimental.pallas import tpu as pltpu
from jax.experimental.pallas import tpu_sc as plsc

N = 10000
E = 320000
D = 128
NC = 2
NS = 16
NW = NC * NS
E_PER_TILE = E // NW
BATCH = 80
N_BATCH = E_PER_TILE // BATCH
CH = 80
NCHUNK = N // CH
NCHUNK_IT = (NCHUNK + NS - 1) // NS
DW = 16

_mesh = plsc.VectorSubcoreMesh(core_axis_name="c", subcore_axis_name="s")


@functools.partial(
    pl.kernel, mesh=_mesh,
    out_type=jax.ShapeDtypeStruct((NC, N, DW), jnp.float32),
    scratch_types=[
        pltpu.VMEM((BATCH,), jnp.int32),
        pltpu.VMEM((BATCH, DW), jnp.float32),
        pltpu.VMEM((CH, DW), jnp.float32),
        pltpu.VMEM_SHARED((N, DW), jnp.float32),
        pltpu.SemaphoreType.DMA,
    ],
)
def _deg_kernel(dst_hbm, out_hbm, idx_v, ones_v, zbuf_v, acc_sh, sem):
    c = lax.axis_index("c")
    s = lax.axis_index("s")
    t = c * NS + s

    def fill_ones(r, carry):
        ones_v[r, :] = jnp.ones((DW,), jnp.float32)
        return carry

    lax.fori_loop(0, BATCH, fill_ones, 0)

    def fill_zero(r, carry):
        zbuf_v[r, :] = jnp.zeros((DW,), jnp.float32)
        return carry

    lax.fori_loop(0, CH, fill_zero, 0)

    def zero_chunk(jj, carry):
        j = s + jj * NS

        @pl.when(j < NCHUNK)
        def _():
            pltpu.sync_copy(zbuf_v, acc_sh.at[pl.ds(j * CH, CH)])

        return carry

    lax.fori_loop(0, NCHUNK_IT, zero_chunk, 0)
    plsc.subcore_barrier()

    def body(i, carry):
        base = t * E_PER_TILE + i * BATCH
        pltpu.sync_copy(dst_hbm.at[pl.ds(base, BATCH)], idx_v)
        pltpu.sync_copy(ones_v, acc_sh.at[idx_v], add=True)
        return carry

    lax.fori_loop(0, N_BATCH, body, 0)
    plsc.subcore_barrier()

    def out_chunk(jj, carry):
        j = s + jj * NS

        @pl.when(j < NCHUNK)
        def _():
            pltpu.sync_copy(acc_sh.at[pl.ds(j * CH, CH)], zbuf_v)
            pltpu.sync_copy(zbuf_v, out_hbm.at[c, pl.ds(j * CH, CH)])

        return carry

    lax.fori_loop(0, NCHUNK_IT, out_chunk, 0)


@functools.partial(
    pl.kernel, mesh=_mesh,
    out_type=jax.ShapeDtypeStruct((NC, N, D), jnp.float32),
    scratch_types=[
        pltpu.VMEM((BATCH,), jnp.int32),
        pltpu.VMEM((BATCH,), jnp.int32),
        pltpu.VMEM((BATCH, D), jnp.float32),
        pltpu.VMEM((CH, D), jnp.float32),
        pltpu.VMEM_SHARED((N, D), jnp.float32),
        pltpu.SemaphoreType.DMA,
    ],
)
def _agg_kernel(g_hbm, src_hbm, dst_hbm, out_hbm, src_v, dst_v, rows_v, zbuf_v,
                acc_sh, sem):
    c = lax.axis_index("c")
    s = lax.axis_index("s")
    t = c * NS + s

    def zrow(r, carry):
        for k in range(D // 16):
            zbuf_v[r, pl.ds(k * 16, 16)] = jnp.zeros((16,), jnp.float32)
        return carry

    lax.fori_loop(0, CH, zrow, 0)

    def zero_chunk(jj, carry):
        j = s + jj * NS

        @pl.when(j < NCHUNK)
        def _():
            pltpu.sync_copy(zbuf_v, acc_sh.at[pl.ds(j * CH, CH)])

        return carry

    lax.fori_loop(0, NCHUNK_IT, zero_chunk, 0)
    plsc.subcore_barrier()

    def body(i, carry):
        base = t * E_PER_TILE + i * BATCH
        pltpu.sync_copy(src_hbm.at[pl.ds(base, BATCH)], src_v)
        pltpu.sync_copy(dst_hbm.at[pl.ds(base, BATCH)], dst_v)
        pltpu.async_copy(g_hbm.at[src_v], rows_v, sem).wait()
        pltpu.sync_copy(rows_v, acc_sh.at[dst_v], add=True)
        return carry

    lax.fori_loop(0, N_BATCH, body, 0)
    plsc.subcore_barrier()

    def out_chunk(jj, carry):
        j = s + jj * NS

        @pl.when(j < NCHUNK)
        def _():
            pltpu.sync_copy(acc_sh.at[pl.ds(j * CH, CH)], zbuf_v)
            pltpu.sync_copy(zbuf_v, out_hbm.at[c, pl.ds(j * CH, CH)])

        return carry

    lax.fori_loop(0, NCHUNK_IT, out_chunk, 0)


@functools.partial(
    pl.kernel, mesh=_mesh,
    out_type=jax.ShapeDtypeStruct((E, D), jnp.float32),
    scratch_types=[
        pltpu.VMEM((BATCH,), jnp.int32),
        pltpu.VMEM((BATCH,), jnp.int32),
        pltpu.VMEM((BATCH, D), jnp.float32),
        pltpu.VMEM((BATCH, D), jnp.float32),
        pltpu.SemaphoreType.DMA,
    ],
)
def _edge_kernel(a_hbm, b_hbm, src_hbm, dst_hbm, z_hbm, src_v, dst_v, va, vb,
                 sem):
    c = lax.axis_index("c")
    s = lax.axis_index("s")
    t = c * NS + s

    def body(i, carry):
        base = t * E_PER_TILE + i * BATCH
        pltpu.sync_copy(src_hbm.at[pl.ds(base, BATCH)], src_v)
        pltpu.sync_copy(dst_hbm.at[pl.ds(base, BATCH)], dst_v)
        cp_a = pltpu.async_copy(a_hbm.at[src_v], va, sem)
        cp_b = pltpu.async_copy(b_hbm.at[dst_v], vb, sem)
        cp_a.wait()
        cp_b.wait()

        def addrow(r, carry2):
            for k in range(D // 16):
                sl = pl.ds(k * 16, 16)
                va[r, sl] = jnp.maximum(va[r, sl] + vb[r, sl], 0.0)
            return carry2

        lax.fori_loop(0, BATCH, addrow, 0)
        pltpu.sync_copy(va, z_hbm.at[pl.ds(base, BATCH)])
        return carry

    lax.fori_loop(0, N_BATCH, body, 0)


BN = 1000
BE = 8000


def _prep1_body(degp_ref, x_ref, w_ref, g_ref, d_ref):
    p = degp_ref[0] + degp_ref[1]
    dd = lax.rsqrt(p + 1.0)
    h = jnp.dot(x_ref[...], w_ref[...], preferred_element_type=jnp.float32)
    g_ref[...] = h * dd
    d_ref[...] = dd


def _prep1(degp1, x, W1):
    return pl.pallas_call(
        _prep1_body,
        grid=(N // BN,),
        in_specs=[
            pl.BlockSpec((2, BN, 1), lambda i: (0, i, 0)),
            pl.BlockSpec((BN, D), lambda i: (i, 0)),
            pl.BlockSpec((D, D), lambda i: (0, 0)),
        ],
        out_specs=[
            pl.BlockSpec((BN, D), lambda i: (i, 0)),
            pl.BlockSpec((BN, 1), lambda i: (i, 0)),
        ],
        out_shape=[
            jax.ShapeDtypeStruct((N, D), jnp.float32),
            jax.ShapeDtypeStruct((N, 1), jnp.float32),
        ],
    )(degp1, x, W1)


def _mid_body(aggp_ref, g_ref, d_ref, b_ref, w_ref, out_ref):
    ssum = aggp_ref[0] + aggp_ref[1] + g_ref[...]
    h = jnp.maximum(d_ref[...] * ssum + b_ref[...], 0.0)
    out_ref[...] = jnp.dot(h, w_ref[...],
                           preferred_element_type=jnp.float32) * d_ref[...]


def _mid(aggp, g, d, b, W):
    return pl.pallas_call(
        _mid_body,
        grid=(N // BN,),
        in_specs=[
            pl.BlockSpec((2, BN, D), lambda i: (0, i, 0)),
            pl.BlockSpec((BN, D), lambda i: (i, 0)),
            pl.BlockSpec((BN, 1), lambda i: (i, 0)),
            pl.BlockSpec((1, D), lambda i: (0, 0)),
            pl.BlockSpec((D, D), lambda i: (0, 0)),
        ],
        out_specs=pl.BlockSpec((BN, D), lambda i: (i, 0)),
        out_shape=jax.ShapeDtypeStruct((N, D), jnp.float32),
    )(aggp, g, d, b, W)


def _fin_body(aggp_ref, g_ref, d_ref, b_ref, wa_ref, wb_ref, bc1_ref, a_ref,
              bo_ref):
    ssum = aggp_ref[0] + aggp_ref[1] + g_ref[...]
    h = jnp.maximum(d_ref[...] * ssum + b_ref[...], 0.0)
    a_ref[...] = jnp.dot(h, wa_ref[...],
                         preferred_element_type=jnp.float32) + bc1_ref[...]
    bo_ref[...] = jnp.dot(h, wb_ref[...], preferred_element_type=jnp.float32)


def _fin(aggp, g, d, b, Wa, Wb, bc1r):
    return pl.pallas_call(
        _fin_body,
        grid=(N // BN,),
        in_specs=[
            pl.BlockSpec((2, BN, D), lambda i: (0, i, 0)),
            pl.BlockSpec((BN, D), lambda i: (i, 0)),
            pl.BlockSpec((BN, 1), lambda i: (i, 0)),
            pl.BlockSpec((1, D), lambda i: (0, 0)),
            pl.BlockSpec((D, D), lambda i: (0, 0)),
            pl.BlockSpec((D, D), lambda i: (0, 0)),
            pl.BlockSpec((1, D), lambda i: (0, 0)),
        ],
        out_specs=[
            pl.BlockSpec((BN, D), lambda i: (i, 0)),
            pl.BlockSpec((BN, D), lambda i: (i, 0)),
        ],
        out_shape=[
            jax.ShapeDtypeStruct((N, D), jnp.float32),
            jax.ShapeDtypeStruct((N, D), jnp.float32),
        ],
    )(aggp, g, d, b, Wa, Wb, bc1r)


def _logits_body(z_ref, w_ref, b_ref, out_ref):
    out_ref[...] = jnp.dot(z_ref[...], w_ref[...],
                           preferred_element_type=jnp.float32) + b_ref[...]


def _logits(z, Wc2, bc2r):
    return pl.pallas_call(
        _logits_body,
        grid=(E // BE,),
        in_specs=[
            pl.BlockSpec((BE, D), lambda i: (i, 0)),
            pl.BlockSpec((D, 2), lambda i: (0, 0)),
            pl.BlockSpec((1, 2), lambda i: (0, 0)),
        ],
        out_specs=pl.BlockSpec((BE, 2), lambda i: (i, 0)),
        out_shape=jax.ShapeDtypeStruct((E, 2), jnp.float32),
    )(z, Wc2, bc2r)


def kernel(x, edge_index, W1, b1, W2, b2, Wc1, bc1, Wc2, bc2):
    ei = edge_index.astype(jnp.int32)
    src = ei[0]
    dst = ei[1]

    degp = _deg_kernel(dst)
    degp1 = degp[:, :, :1]
    g1, d = _prep1(degp1, x, W1)

    aggp1 = _agg_kernel(g1, src, dst)
    g2 = _mid(aggp1, g1, d, b1.reshape(1, D), W2)

    aggp2 = _agg_kernel(g2, src, dst)
    A, B = _fin(aggp2, g2, d, b2.reshape(1, D), Wc1[:D], Wc1[D:],
                bc1.reshape(1, D))

    z = _edge_kernel(A, B, src, dst)
    return _logits(z, Wc2, bc2.reshape(1, 2))


# speedup vs baseline: 15.2833x; 1.8951x over previous
import functools

import jax
import jax.numpy as jnp
from jax import lax
from jax.experimental import pallas as pl
from jax.experimental.pallas import tpu as pltpu
from jax.experimental.pallas import tpu_sc as plsc

N = 10000
E = 320000
D = 128
NC = 2
NS = 16
NW = NC * NS
E_PER_TILE = E // NW
BATCH = 80
N_BATCH = E_PER_TILE // BATCH
CH = 80
NCHUNK = N // CH
NCHUNK_IT = (NCHUNK + NS - 1) // NS
DW = 16

_mesh = plsc.VectorSubcoreMesh(core_axis_name="c", subcore_axis_name="s")


@functools.partial(
    pl.kernel, mesh=_mesh,
    out_type=jax.ShapeDtypeStruct((NC, N, DW), jnp.float32),
    scratch_types=[
        pltpu.VMEM((BATCH,), jnp.int32),
        pltpu.VMEM((BATCH,), jnp.int32),
        pltpu.VMEM((BATCH, DW), jnp.float32),
        pltpu.VMEM((CH, DW), jnp.float32),
        pltpu.VMEM_SHARED((N, DW), jnp.float32),
        pltpu.SemaphoreType.DMA,
        pltpu.SemaphoreType.DMA,
    ],
)
def _deg_kernel(dst_hbm, out_hbm, idx0, idx1, ones_v, zbuf_v, acc_sh, sem0,
                sem1):
    c = lax.axis_index("c")
    s = lax.axis_index("s")
    t = c * NS + s

    def fill_ones(r, carry):
        ones_v[r, :] = jnp.ones((DW,), jnp.float32)
        return carry

    lax.fori_loop(0, BATCH, fill_ones, 0)

    def fill_zero(r, carry):
        zbuf_v[r, :] = jnp.zeros((DW,), jnp.float32)
        return carry

    lax.fori_loop(0, CH, fill_zero, 0)

    def zero_chunk(jj, carry):
        j = s + jj * NS

        @pl.when(j < NCHUNK)
        def _():
            pltpu.sync_copy(zbuf_v, acc_sh.at[pl.ds(j * CH, CH)])

        return carry

    lax.fori_loop(0, NCHUNK_IT, zero_chunk, 0)
    plsc.subcore_barrier()

    def body(i, carry):
        base = t * E_PER_TILE + i * BATCH
        pltpu.sync_copy(dst_hbm.at[pl.ds(base, BATCH)], idx0)
        pltpu.sync_copy(ones_v, acc_sh.at[idx0], add=True)
        return carry

    lax.fori_loop(0, N_BATCH, body, 0)
    plsc.subcore_barrier()

    def out_chunk(jj, carry):
        j = s + jj * NS

        @pl.when(j < NCHUNK)
        def _():
            pltpu.sync_copy(acc_sh.at[pl.ds(j * CH, CH)], zbuf_v)
            pltpu.sync_copy(zbuf_v, out_hbm.at[c, pl.ds(j * CH, CH)])

        return carry

    lax.fori_loop(0, NCHUNK_IT, out_chunk, 0)


@functools.partial(
    pl.kernel, mesh=_mesh,
    out_type=jax.ShapeDtypeStruct((NC, N, D), jnp.float32),
    scratch_types=[
        pltpu.VMEM((E_PER_TILE,), jnp.int32),
        pltpu.VMEM((N_BATCH, BATCH), jnp.int32),
        pltpu.VMEM((BATCH, D), jnp.float32),
        pltpu.VMEM((BATCH, D), jnp.float32),
        pltpu.VMEM_SHARED((N, D), jnp.float32),
        pltpu.SemaphoreType.DMA,
        pltpu.SemaphoreType.DMA,
        pltpu.SemaphoreType.DMA,
    ],
)
def _agg_kernel(g_hbm, src2_hbm, dst3_hbm, out_hbm, src_v, dst_v, rows0, rows1,
                acc_sh, semi, sem0, sem1):
    c = lax.axis_index("c")
    s = lax.axis_index("s")
    t = c * NS + s

    pltpu.async_copy(src2_hbm.at[t], src_v, semi)
    pltpu.async_copy(dst3_hbm.at[t], dst_v, semi)

    def zrow(r, carry):
        for k in range(D // 16):
            rows0[r, pl.ds(k * 16, 16)] = jnp.zeros((16,), jnp.float32)
        return carry

    lax.fori_loop(0, CH, zrow, 0)

    def zero_chunk(jj, carry):
        j = s + jj * NS

        @pl.when(j < NCHUNK)
        def _():
            pltpu.sync_copy(rows0, acc_sh.at[pl.ds(j * CH, CH)])

        return carry

    lax.fori_loop(0, NCHUNK_IT, zero_chunk, 0)
    pltpu.make_async_copy(src2_hbm.at[t], src_v, semi).wait()
    pltpu.make_async_copy(dst3_hbm.at[t], dst_v, semi).wait()
    plsc.subcore_barrier()

    rows = (rows0, rows1)
    sems = (sem0, sem1)

    def issue(a, slot):
        pltpu.async_copy(g_hbm.at[src_v.at[pl.ds(a * BATCH, BATCH)]],
                         rows[slot], sems[slot])

    def drain(slot):
        pltpu.make_async_copy(g_hbm.at[src_v.at[pl.ds(0, BATCH)]],
                              rows[slot], sems[slot]).wait()

    issue(0, 0)
    issue(1, 1)

    def body(i2, carry):
        for slot in range(2):
            a = 2 * i2 + slot

            @pl.when(a < N_BATCH)
            def _():
                drain(slot)
                pltpu.sync_copy(rows[slot], acc_sh.at[dst_v.at[a]], add=True)

                @pl.when(a + 2 < N_BATCH)
                def _():
                    issue(a + 2, slot)

        return carry

    lax.fori_loop(0, (N_BATCH + 1) // 2, body, 0)
    plsc.subcore_barrier()

    def out_chunk(jj, carry):
        j = s + jj * NS

        @pl.when(j < NCHUNK)
        def _():
            pltpu.sync_copy(acc_sh.at[pl.ds(j * CH, CH)],
                            out_hbm.at[c, pl.ds(j * CH, CH)])

        return carry

    lax.fori_loop(0, NCHUNK_IT, out_chunk, 0)


@functools.partial(
    pl.kernel, mesh=_mesh,
    out_type=jax.ShapeDtypeStruct((E, D), jnp.float32),
    scratch_types=[
        pltpu.VMEM((E_PER_TILE,), jnp.int32),
        pltpu.VMEM((E_PER_TILE,), jnp.int32),
        pltpu.VMEM((BATCH, D), jnp.float32),
        pltpu.VMEM((BATCH, D), jnp.float32),
        pltpu.VMEM((BATCH, D), jnp.float32),
        pltpu.VMEM((BATCH, D), jnp.float32),
        pltpu.VMEM((BATCH, D), jnp.float32),
        pltpu.VMEM((BATCH, D), jnp.float32),
        pltpu.SemaphoreType.DMA,
        pltpu.SemaphoreType.DMA,
        pltpu.SemaphoreType.DMA,
        pltpu.SemaphoreType.DMA,
        pltpu.SemaphoreType.DMA,
        pltpu.SemaphoreType.DMA,
        pltpu.SemaphoreType.DMA,
    ],
)
def _edge_kernel(a_hbm, b_hbm, src2_hbm, dst2_hbm, z_hbm, src_v, dst_v,
                 va0, va1, vb0, vb1, w0, w1,
                 semi, semA0, semA1, semB0, semB1, semS0, semS1):
    c = lax.axis_index("c")
    s = lax.axis_index("s")
    t = c * NS + s

    pltpu.async_copy(src2_hbm.at[t], src_v, semi)
    pltpu.async_copy(dst2_hbm.at[t], dst_v, semi)
    pltpu.make_async_copy(src2_hbm.at[t], src_v, semi).wait()
    pltpu.make_async_copy(dst2_hbm.at[t], dst_v, semi).wait()

    va = (va0, va1)
    vb = (vb0, vb1)
    w = (w0, w1)
    semA = (semA0, semA1)
    semB = (semB0, semB1)
    semS = (semS0, semS1)

    def issue(a, slot):
        pltpu.async_copy(a_hbm.at[src_v.at[pl.ds(a * BATCH, BATCH)]],
                         va[slot], semA[slot])
        pltpu.async_copy(b_hbm.at[dst_v.at[pl.ds(a * BATCH, BATCH)]],
                         vb[slot], semB[slot])

    def drain_g(slot):
        pltpu.make_async_copy(a_hbm.at[src_v.at[pl.ds(0, BATCH)]],
                              va[slot], semA[slot]).wait()
        pltpu.make_async_copy(b_hbm.at[dst_v.at[pl.ds(0, BATCH)]],
                              vb[slot], semB[slot]).wait()

    issue(0, 0)
    issue(1, 1)

    def body(i2, carry):
        for slot in range(2):
            a = 2 * i2 + slot

            @pl.when(a < N_BATCH)
            def _():
                drain_g(slot)

                @pl.when(a >= 2)
                def _():
                    pltpu.make_async_copy(
                        w[slot], z_hbm.at[pl.ds(0, BATCH)], semS[slot]).wait()

                def addrow(r, carry2):
                    for k in range(D // 16):
                        sl = pl.ds(k * 16, 16)
                        w[slot][r, sl] = jnp.maximum(
                            va[slot][r, sl] + vb[slot][r, sl], 0.0)
                    return carry2

                lax.fori_loop(0, BATCH, addrow, 0)
                base = t * E_PER_TILE + a * BATCH
                pltpu.async_copy(w[slot], z_hbm.at[pl.ds(base, BATCH)],
                                 semS[slot])

                @pl.when(a + 2 < N_BATCH)
                def _():
                    issue(a + 2, slot)

        return carry

    lax.fori_loop(0, (N_BATCH + 1) // 2, body, 0)
    pltpu.make_async_copy(w0, z_hbm.at[pl.ds(0, BATCH)], semS0).wait()
    pltpu.make_async_copy(w1, z_hbm.at[pl.ds(0, BATCH)], semS1).wait()


BN = 1000
BE = 8000


def _prep1_body(degp_ref, x_ref, w_ref, g_ref, d_ref):
    p = degp_ref[0] + degp_ref[1]
    dd = lax.rsqrt(p + 1.0)
    h = jnp.dot(x_ref[...], w_ref[...], preferred_element_type=jnp.float32)
    g_ref[...] = h * dd
    d_ref[...] = dd


def _prep1(degp1, x, W1):
    return pl.pallas_call(
        _prep1_body,
        grid=(N // BN,),
        in_specs=[
            pl.BlockSpec((2, BN, 1), lambda i: (0, i, 0)),
            pl.BlockSpec((BN, D), lambda i: (i, 0)),
            pl.BlockSpec((D, D), lambda i: (0, 0)),
        ],
        out_specs=[
            pl.BlockSpec((BN, D), lambda i: (i, 0)),
            pl.BlockSpec((BN, 1), lambda i: (i, 0)),
        ],
        out_shape=[
            jax.ShapeDtypeStruct((N, D), jnp.float32),
            jax.ShapeDtypeStruct((N, 1), jnp.float32),
        ],
    )(degp1, x, W1)


def _mid_body(aggp_ref, g_ref, d_ref, b_ref, w_ref, out_ref):
    ssum = aggp_ref[0] + aggp_ref[1] + g_ref[...]
    h = jnp.maximum(d_ref[...] * ssum + b_ref[...], 0.0)
    out_ref[...] = jnp.dot(h, w_ref[...],
                           preferred_element_type=jnp.float32) * d_ref[...]


def _mid(aggp, g, d, b, W):
    return pl.pallas_call(
        _mid_body,
        grid=(N // BN,),
        in_specs=[
            pl.BlockSpec((2, BN, D), lambda i: (0, i, 0)),
            pl.BlockSpec((BN, D), lambda i: (i, 0)),
            pl.BlockSpec((BN, 1), lambda i: (i, 0)),
            pl.BlockSpec((1, D), lambda i: (0, 0)),
            pl.BlockSpec((D, D), lambda i: (0, 0)),
        ],
        out_specs=pl.BlockSpec((BN, D), lambda i: (i, 0)),
        out_shape=jax.ShapeDtypeStruct((N, D), jnp.float32),
    )(aggp, g, d, b, W)


def _fin_body(aggp_ref, g_ref, d_ref, b_ref, wa_ref, wb_ref, bc1_ref, a_ref,
              bo_ref):
    ssum = aggp_ref[0] + aggp_ref[1] + g_ref[...]
    h = jnp.maximum(d_ref[...] * ssum + b_ref[...], 0.0)
    a_ref[...] = jnp.dot(h, wa_ref[...],
                         preferred_element_type=jnp.float32) + bc1_ref[...]
    bo_ref[...] = jnp.dot(h, wb_ref[...], preferred_element_type=jnp.float32)


def _fin(aggp, g, d, b, Wa, Wb, bc1r):
    return pl.pallas_call(
        _fin_body,
        grid=(N // BN,),
        in_specs=[
            pl.BlockSpec((2, BN, D), lambda i: (0, i, 0)),
            pl.BlockSpec((BN, D), lambda i: (i, 0)),
            pl.BlockSpec((BN, 1), lambda i: (i, 0)),
            pl.BlockSpec((1, D), lambda i: (0, 0)),
            pl.BlockSpec((D, D), lambda i: (0, 0)),
            pl.BlockSpec((D, D), lambda i: (0, 0)),
            pl.BlockSpec((1, D), lambda i: (0, 0)),
        ],
        out_specs=[
            pl.BlockSpec((BN, D), lambda i: (i, 0)),
            pl.BlockSpec((BN, D), lambda i: (i, 0)),
        ],
        out_shape=[
            jax.ShapeDtypeStruct((N, D), jnp.float32),
            jax.ShapeDtypeStruct((N, D), jnp.float32),
        ],
    )(aggp, g, d, b, Wa, Wb, bc1r)


def _logits_body(z_ref, w_ref, b_ref, out_ref):
    out_ref[...] = jnp.dot(z_ref[...], w_ref[...],
                           preferred_element_type=jnp.float32) + b_ref[...]


def _logits(z, Wc2, bc2r):
    return pl.pallas_call(
        _logits_body,
        grid=(E // BE,),
        in_specs=[
            pl.BlockSpec((BE, D), lambda i: (i, 0)),
            pl.BlockSpec((D, 2), lambda i: (0, 0)),
            pl.BlockSpec((1, 2), lambda i: (0, 0)),
        ],
        out_specs=pl.BlockSpec((BE, 2), lambda i: (i, 0)),
        out_shape=jax.ShapeDtypeStruct((E, 2), jnp.float32),
    )(z, Wc2, bc2r)


def kernel(x, edge_index, W1, b1, W2, b2, Wc1, bc1, Wc2, bc2):
    ei = edge_index.astype(jnp.int32)
    src = ei[0]
    dst = ei[1]
    src2 = src.reshape(NW, E_PER_TILE)
    dst2 = dst.reshape(NW, E_PER_TILE)
    dst3 = dst.reshape(NW, N_BATCH, BATCH)

    degp = _deg_kernel(dst)
    degp1 = degp[:, :, :1]
    g1, d = _prep1(degp1, x, W1)

    aggp1 = _agg_kernel(g1, src2, dst3)
    g2 = _mid(aggp1, g1, d, b1.reshape(1, D), W2)

    aggp2 = _agg_kernel(g2, src2, dst3)
    A, B = _fin(aggp2, g2, d, b2.reshape(1, D), Wc1[:D], Wc1[D:],
                bc1.reshape(1, D))

    z = _edge_kernel(A, B, src2, dst2)
    return _logits(z, Wc2, bc2.reshape(1, 2))
